# baseline (speedup 1.0000x reference)
import jax
import jax.numpy as jnp
from jax.experimental import pallas as pl

_B, _N, _M, _D = 8, 2048, 2048, 3


def _bf(a):
    return a.astype(jnp.bfloat16).astype(jnp.float32)


def _chamfer_batch(x_ref, y_ref, d1_ref, d2_ref):
    xb = x_ref[0]
    yb = y_ref[0]
    nx = jnp.sum(xb * xb, axis=0, keepdims=True)
    ny = jnp.sum(yb * yb, axis=0, keepdims=True)
    y2 = -2.0 * yb

    hx = _bf(xb)
    lx = _bf(xb - hx)
    hy = _bf(y2)
    ly = _bf(y2 - hy)
    nxh = _bf(nx)
    nxl = _bf(nx - nxh)
    nxll = _bf(nx - nxh - nxl)
    nyh = _bf(ny)
    nyl = _bf(ny - nyh)
    nyll = _bf(ny - nyh - nyl)
    ones_n = jnp.ones((3, _N), jnp.float32)
    ones_m = jnp.ones((3, _M), jnp.float32)
    zeros_n = jnp.zeros((1, _N), jnp.float32)
    zeros_m = jnp.zeros((1, _M), jnp.float32)

    lhs = jnp.concatenate(
        [hx, hx, lx, nxh, nxl, nxll, ones_n, zeros_n],
        axis=0).astype(jnp.bfloat16)
    rhs = jnp.concatenate(
        [hy, ly, hy, ones_m, nyh, nyl, nyll, zeros_m],
        axis=0).astype(jnp.bfloat16)
    d = jax.lax.dot_general(
        lhs, rhs, dimension_numbers=(((0,), (0,)), ((), ())),
        preferred_element_type=jnp.float32)
    d1_ref[0] = jnp.min(d, axis=1, keepdims=True)
    d2_ref[0, 0, :] = jnp.min(d, axis=0)


@jax.jit
def kernel(input1, input2):
    x = jnp.transpose(input1, (0, 2, 1))
    y = jnp.transpose(input2, (0, 2, 1))
    d1, d2 = pl.pallas_call(
        _chamfer_batch,
        grid=(_B,),
        in_specs=[
            pl.BlockSpec((1, _D, _N), lambda b: (b, 0, 0)),
            pl.BlockSpec((1, _D, _M), lambda b: (b, 0, 0)),
        ],
        out_specs=[
            pl.BlockSpec((1, _N, 1), lambda b: (b, 0, 0)),
            pl.BlockSpec((1, 1, _M), lambda b: (b, 0, 0)),
        ],
        out_shape=[
            jax.ShapeDtypeStruct((_B, _N, 1), jnp.float32),
            jax.ShapeDtypeStruct((_B, 1, _M), jnp.float32),
        ],
    )(x, y)
    return (d1[:, :, 0], d2[:, 0, :])


# speedup vs baseline: 1.0319x; 1.0319x over previous
import jax
import jax.numpy as jnp
from jax.experimental import pallas as pl

_B, _N, _M, _D = 8, 2048, 2048, 3
_BB = 2


def _bf(a):
    return a.astype(jnp.bfloat16).astype(jnp.float32)


def _chamfer_batch(x_ref, y_ref, d1_ref, d2_ref):
    for b in range(_BB):
        xb = x_ref[b]
        yb = y_ref[b]
        nx = jnp.sum(xb * xb, axis=0, keepdims=True)
        ny = jnp.sum(yb * yb, axis=0, keepdims=True)
        y2 = -2.0 * yb

        hx = _bf(xb)
        lx = _bf(xb - hx)
        hy = _bf(y2)
        ly = _bf(y2 - hy)
        nxh = _bf(nx)
        nxl = _bf(nx - nxh)
        nxll = _bf(nx - nxh - nxl)
        nyh = _bf(ny)
        nyl = _bf(ny - nyh)
        nyll = _bf(ny - nyh - nyl)
        ones_n = jnp.ones((3, _N), jnp.float32)
        ones_m = jnp.ones((3, _M), jnp.float32)
        zeros_n = jnp.zeros((1, _N), jnp.float32)
        zeros_m = jnp.zeros((1, _M), jnp.float32)

        lhs = jnp.concatenate(
            [hx, hx, lx, nxh, nxl, nxll, ones_n, zeros_n],
            axis=0).astype(jnp.bfloat16)
        rhs = jnp.concatenate(
            [hy, ly, hy, ones_m, nyh, nyl, nyll, zeros_m],
            axis=0).astype(jnp.bfloat16)
        d = jax.lax.dot_general(
            lhs, rhs, dimension_numbers=(((0,), (0,)), ((), ())),
            preferred_element_type=jnp.float32)
        d1_ref[b] = jnp.min(d, axis=1, keepdims=True)
        d2_ref[b, 0, :] = jnp.min(d, axis=0)


@jax.jit
def kernel(input1, input2):
    x = jnp.transpose(input1, (0, 2, 1))
    y = jnp.transpose(input2, (0, 2, 1))
    d1, d2 = pl.pallas_call(
        _chamfer_batch,
        grid=(_B // _BB,),
        in_specs=[
            pl.BlockSpec((_BB, _D, _N), lambda b: (b, 0, 0)),
            pl.BlockSpec((_BB, _D, _M), lambda b: (b, 0, 0)),
        ],
        out_specs=[
            pl.BlockSpec((_BB, _N, 1), lambda b: (b, 0, 0)),
            pl.BlockSpec((_BB, 1, _M), lambda b: (b, 0, 0)),
        ],
        out_shape=[
            jax.ShapeDtypeStruct((_B, _N, 1), jnp.float32),
            jax.ShapeDtypeStruct((_B, 1, _M), jnp.float32),
        ],
    )(x, y)
    return (d1[:, :, 0], d2[:, 0, :])
